# baseline (device time: 83323 ns/iter reference)
import jax
import jax.numpy as jnp
from jax import lax
from jax.experimental import pallas as pl
from jax.experimental.pallas import tpu as pltpu

B, S, H, Dh, Dr = 2, 256, 16, 64, 32
D = 1024
SCALE = (Dh + Dr) ** -0.5


def kernel(x, Wdkv, Wuk, Wuv, Wq, Wqr, Wkr, Wo):
    def body(x_ref, wdkv_ref, wuk_ref, wuv_ref, wq_ref, wqr_ref, wkr_ref,
             wo_ref, out_ref, kv_send, kv_recv, o_scr, send_sem, recv_sem):
        my_x = lax.axis_index("x")
        my_y = lax.axis_index("y")
        my_z = lax.axis_index("z")
        nbr = (my_x, 1 - my_y, my_z)

        for b in range(B):
            xb = x_ref[b]
            cb = jnp.dot(xb, wdkv_ref[...], preferred_element_type=jnp.float32)
            kv_send[b, :, 0:D] = jnp.dot(
                cb, wuk_ref[...], preferred_element_type=jnp.float32)
            kv_send[b, :, D:2 * D] = jnp.dot(
                cb, wuv_ref[...], preferred_element_type=jnp.float32)

        barrier_sem = pltpu.get_barrier_semaphore()
        pl.semaphore_signal(barrier_sem, inc=1, device_id=nbr,
                            device_id_type=pl.DeviceIdType.MESH)
        pl.semaphore_wait(barrier_sem, 1)

        rdma = pltpu.make_async_remote_copy(
            src_ref=kv_send, dst_ref=kv_recv,
            send_sem=send_sem, recv_sem=recv_sem,
            device_id=nbr, device_id_type=pl.DeviceIdType.MESH)
        rdma.start()
        rdma.wait()

        for b in range(B):
            xb = x_ref[b]
            qb = jnp.dot(xb, wq_ref[...], preferred_element_type=jnp.float32)
            qrb = jnp.dot(xb, wqr_ref[...], preferred_element_type=jnp.float32)
            krb = jnp.dot(xb, wkr_ref[...], preferred_element_type=jnp.float32)
            kb = kv_send[b, :, 0:D] + kv_recv[b, :, 0:D]
            vb = kv_send[b, :, D:2 * D] + kv_recv[b, :, D:2 * D]
            for h in range(H):
                qh = qb[:, h * Dh:(h + 1) * Dh]
                kh = kb[:, h * Dh:(h + 1) * Dh]
                qrh = qrb[:, h * Dr:(h + 1) * Dr]
                s1 = lax.dot_general(qh, kh, (((1,), (1,)), ((), ())),
                                     preferred_element_type=jnp.float32)
                s2 = lax.dot_general(qrh, krb, (((1,), (1,)), ((), ())),
                                     preferred_element_type=jnp.float32)
                sc = (s1 + s2) * SCALE
                m = jnp.max(sc, axis=-1, keepdims=True)
                p = jnp.exp(sc - m)
                p = p / jnp.sum(p, axis=-1, keepdims=True)
                vh = vb[:, h * Dh:(h + 1) * Dh]
                o_scr[:, h * Dh:(h + 1) * Dh] = jnp.dot(
                    p, vh, preferred_element_type=jnp.float32)
            out_ref[b] = jnp.dot(o_scr[...], wo_ref[...],
                                 preferred_element_type=jnp.float32)

    return pl.pallas_call(
        body,
        out_shape=jax.ShapeDtypeStruct((B, S, D), jnp.float32),
        in_specs=[pl.BlockSpec(memory_space=pltpu.VMEM)] * 8,
        out_specs=pl.BlockSpec(memory_space=pltpu.VMEM),
        scratch_shapes=[
            pltpu.VMEM((B, S, 2 * D), jnp.float32),
            pltpu.VMEM((B, S, 2 * D), jnp.float32),
            pltpu.VMEM((S, D), jnp.float32),
            pltpu.SemaphoreType.DMA,
            pltpu.SemaphoreType.DMA,
        ],
        compiler_params=pltpu.CompilerParams(collective_id=0),
    )(x, Wdkv, Wuk, Wuv, Wq, Wqr, Wkr, Wo)


# device time: 44503 ns/iter; 1.8723x vs baseline; 1.8723x over previous
import jax
import jax.numpy as jnp
from jax import lax
from jax.experimental import pallas as pl
from jax.experimental.pallas import tpu as pltpu

B, S, H, Dh, Dr = 2, 256, 16, 64, 32
D = 1024
DC = 64
SCALE = (Dh + Dr) ** -0.5


def kernel(x, Wdkv, Wuk, Wuv, Wq, Wqr, Wkr, Wo):
    def body(x_ref, wdkv_ref, wuk_ref, wuv_ref, wq_ref, wqr_ref, wkr_ref,
             wo_ref, out_ref, c_send, c_recv, wuk_recv, wuv_recv, o_scr,
             send_sems, recv_sems):
        my_x = lax.axis_index("x")
        my_y = lax.axis_index("y")
        my_z = lax.axis_index("z")
        nbr = (my_x, 1 - my_y, my_z)

        for b in range(B):
            c_send[b] = jnp.dot(x_ref[b], wdkv_ref[...],
                                preferred_element_type=jnp.float32)

        barrier_sem = pltpu.get_barrier_semaphore()
        pl.semaphore_signal(barrier_sem, inc=1, device_id=nbr,
                            device_id_type=pl.DeviceIdType.MESH)
        pl.semaphore_wait(barrier_sem, 1)

        rdmas = []
        for i, (src, dst) in enumerate(
                [(c_send, c_recv), (wuk_ref, wuk_recv), (wuv_ref, wuv_recv)]):
            r = pltpu.make_async_remote_copy(
                src_ref=src, dst_ref=dst,
                send_sem=send_sems.at[i], recv_sem=recv_sems.at[i],
                device_id=nbr, device_id_type=pl.DeviceIdType.MESH)
            r.start()
            rdmas.append(r)

        qs, qrs, krs = [], [], []
        for b in range(B):
            xb = x_ref[b]
            qs.append(jnp.dot(xb, wq_ref[...],
                              preferred_element_type=jnp.float32))
            qrs.append(jnp.dot(xb, wqr_ref[...],
                               preferred_element_type=jnp.float32))
            krs.append(jnp.dot(xb, wkr_ref[...],
                               preferred_element_type=jnp.float32))

        for r in rdmas:
            r.wait()

        for b in range(B):
            qb, qrb, krb = qs[b], qrs[b], krs[b]
            kb = (jnp.dot(c_send[b], wuk_ref[...],
                          preferred_element_type=jnp.float32)
                  + jnp.dot(c_recv[b], wuk_recv[...],
                            preferred_element_type=jnp.float32))
            vb = (jnp.dot(c_send[b], wuv_ref[...],
                          preferred_element_type=jnp.float32)
                  + jnp.dot(c_recv[b], wuv_recv[...],
                            preferred_element_type=jnp.float32))
            for h in range(H):
                qh = qb[:, h * Dh:(h + 1) * Dh]
                kh = kb[:, h * Dh:(h + 1) * Dh]
                qrh = qrb[:, h * Dr:(h + 1) * Dr]
                s1 = lax.dot_general(qh, kh, (((1,), (1,)), ((), ())),
                                     preferred_element_type=jnp.float32)
                s2 = lax.dot_general(qrh, krb, (((1,), (1,)), ((), ())),
                                     preferred_element_type=jnp.float32)
                sc = (s1 + s2) * SCALE
                m = jnp.max(sc, axis=-1, keepdims=True)
                p = jnp.exp(sc - m)
                p = p / jnp.sum(p, axis=-1, keepdims=True)
                vh = vb[:, h * Dh:(h + 1) * Dh]
                o_scr[:, h * Dh:(h + 1) * Dh] = jnp.dot(
                    p, vh, preferred_element_type=jnp.float32)
            out_ref[b] = jnp.dot(o_scr[...], wo_ref[...],
                                 preferred_element_type=jnp.float32)

    return pl.pallas_call(
        body,
        out_shape=jax.ShapeDtypeStruct((B, S, D), jnp.float32),
        in_specs=[pl.BlockSpec(memory_space=pltpu.VMEM)] * 8,
        out_specs=pl.BlockSpec(memory_space=pltpu.VMEM),
        scratch_shapes=[
            pltpu.VMEM((B, S, DC), jnp.float32),
            pltpu.VMEM((B, S, DC), jnp.float32),
            pltpu.VMEM((DC, D), jnp.float32),
            pltpu.VMEM((DC, D), jnp.float32),
            pltpu.VMEM((S, D), jnp.float32),
            pltpu.SemaphoreType.DMA((3,)),
            pltpu.SemaphoreType.DMA((3,)),
        ],
        compiler_params=pltpu.CompilerParams(collective_id=0),
    )(x, Wdkv, Wuk, Wuv, Wq, Wqr, Wkr, Wo)
